# baseline (speedup 1.0000x reference)
import jax
import jax.numpy as jnp
from jax import lax
from jax.experimental import pallas as pl
from jax.experimental.pallas import tpu as pltpu
from jax.experimental.pallas import tpu_sc as plsc

HEIGHT = 32
WIDTH = 32
CH = 192
B = 64
HW = HEIGHT * WIDTH
L = 16

NC = 2
NS = 16
NW = NC * NS
CPW = CH // NW

CB = 1
NBUF = 4
NCHUNK = B // CB
PF = 2


def _sc_body(x_hbm, h_hbm, w_hbm, out_hbm,
             h_v, w_v, pos_v, buf0, buf1, buf2, buf3,
             in_s0, in_s1, in_s2, in_s3, out_s0, out_s1, out_s2, out_s3):
    bufs = (buf0, buf1, buf2, buf3)
    in_sems = (in_s0, in_s1, in_s2, in_s3)
    out_sems = (out_s0, out_s1, out_s2, out_s3)

    wid = lax.axis_index("s") * NC + lax.axis_index("c")
    c0 = wid * CPW

    pltpu.sync_copy(h_hbm, h_v)
    pltpu.sync_copy(w_hbm, w_v)

    lanes = lax.iota(jnp.int32, L)

    def posq(q, carry):
        jj = q // HEIGHT
        h = q % HEIGHT
        cidx = jnp.full((L,), c0 + jj, jnp.int32)
        hvv = plsc.load_gather(h_v, [jnp.full((L,), h, jnp.int32), cidx])
        wv0 = plsc.load_gather(w_v, [lanes, cidx])
        wv1 = plsc.load_gather(w_v, [lanes + L, cidx])
        pos_v[jj, pl.ds(h * WIDTH, L)] = hvv + wv0
        pos_v[jj, pl.ds(h * WIDTH + L, L)] = hvv + wv1
        return carry

    lax.fori_loop(0, CPW * HEIGHT, posq, 0)

    def in_slice(idx):
        return x_hbm.at[pl.ds(idx * CB, CB), pl.ds(c0, CPW), :, :]

    def out_slice(idx):
        return out_hbm.at[pl.ds(idx * CB, CB), pl.ds(c0, CPW), :, :]

    for t in range(NBUF):
        pltpu.async_copy(in_slice(t), bufs[t], in_sems[t])

    def round_body(g, carry):
        for t in range(NBUF):
            idx = g * NBUF + t
            buf = bufs[t]
            pltpu.make_async_copy(in_slice(idx), buf, in_sems[t]).wait()

            def ck(h, kcarry, buf=buf):
                for j in range(CPW):
                    for half in range(2):
                        pv = pos_v[j, pl.ds(h * WIDTH + half * L, L)]
                        for b in range(CB):
                            plsc.addupdate(
                                buf.at[b, j, h, pl.ds(half * L, L)], pv)
                return kcarry

            lax.fori_loop(0, HEIGHT, ck, 0)

            pltpu.async_copy(buf, out_slice(idx), out_sems[t])

            nxt = idx + PF
            t2 = (t + PF) % NBUF

            @pl.when(jnp.logical_and(nxt >= NBUF, nxt < NCHUNK))
            def _refill(nxt=nxt, t2=t2):
                pltpu.make_async_copy(bufs[t2], out_slice(nxt - NBUF),
                                      out_sems[t2]).wait()
                pltpu.async_copy(in_slice(nxt), bufs[t2], in_sems[t2])

        return carry

    lax.fori_loop(0, NCHUNK // NBUF, round_body, 0)

    for idx in range(NCHUNK - NBUF, NCHUNK):
        t = idx % NBUF
        pltpu.make_async_copy(bufs[t], out_slice(idx), out_sems[t]).wait()


def kernel(x, h_emb, w_emb):
    b, c, h, w = x.shape

    mesh = plsc.VectorSubcoreMesh(core_axis_name="c", subcore_axis_name="s")
    run = pl.kernel(
        _sc_body,
        mesh=mesh,
        compiler_params=pltpu.CompilerParams(needs_layout_passes=False),
        out_type=jax.ShapeDtypeStruct((b, c, h, w), jnp.float32),
        scratch_types=[
            pltpu.VMEM((HEIGHT, CH), jnp.float32),
            pltpu.VMEM((WIDTH, CH), jnp.float32),
            pltpu.VMEM((CPW, HW), jnp.float32),
            pltpu.VMEM((CB, CPW, HEIGHT, WIDTH), jnp.float32),
            pltpu.VMEM((CB, CPW, HEIGHT, WIDTH), jnp.float32),
            pltpu.VMEM((CB, CPW, HEIGHT, WIDTH), jnp.float32),
            pltpu.VMEM((CB, CPW, HEIGHT, WIDTH), jnp.float32),
            pltpu.SemaphoreType.DMA,
            pltpu.SemaphoreType.DMA,
            pltpu.SemaphoreType.DMA,
            pltpu.SemaphoreType.DMA,
            pltpu.SemaphoreType.DMA,
            pltpu.SemaphoreType.DMA,
            pltpu.SemaphoreType.DMA,
            pltpu.SemaphoreType.DMA,
        ],
    )
    return run(x, h_emb, w_emb)


# speedup vs baseline: 1.0994x; 1.0994x over previous
import jax
import jax.numpy as jnp
from jax.experimental import pallas as pl
from jax.experimental.pallas import tpu as pltpu

HEIGHT = 32
WIDTH = 32
CH = 192
B = 64

_NB = 2


def _body(x_ref, h_ref, w_ref, o_ref, pos_ref):
    @pl.when(pl.program_id(0) == 0)
    def _build_pos():
        ht = jnp.transpose(h_ref[...])
        wt = jnp.transpose(w_ref[...])
        pos_ref[...] = ht[:, :, None] + wt[:, None, :]

    o_ref[...] = x_ref[...] + pos_ref[...][None]


def kernel(x, h_emb, w_emb):
    b, c, h, w = x.shape
    return pl.pallas_call(
        _body,
        grid=(b // _NB,),
        in_specs=[
            pl.BlockSpec((_NB, c, h, w), lambda i: (i, 0, 0, 0)),
            pl.BlockSpec((HEIGHT, CH), lambda i: (0, 0)),
            pl.BlockSpec((WIDTH, CH), lambda i: (0, 0)),
        ],
        out_specs=pl.BlockSpec((_NB, c, h, w), lambda i: (i, 0, 0, 0)),
        out_shape=jax.ShapeDtypeStruct((b, c, h, w), jnp.float32),
        scratch_shapes=[pltpu.VMEM((CH, HEIGHT, WIDTH), jnp.float32)],
    )(x, h_emb, w_emb)


# speedup vs baseline: 1.2892x; 1.1726x over previous
import jax
import jax.numpy as jnp
from jax import lax
from jax.experimental import pallas as pl
from jax.experimental.pallas import tpu as pltpu
from jax.experimental.pallas import tpu_sc as plsc

HEIGHT = 32
WIDTH = 32
CH = 192
B = 64
HW = HEIGHT * WIDTH
L = 16

NC = 2
NS = 16
NW = NC * NS
CPW = CH // NW
RPW = CPW * HEIGHT

CB = 2
NBUF = 2
NCHUNK = B // CB
PF = 1


def _sc_body(x_hbm, h_hbm, w_hbm, out_hbm,
             h_v, w_v, pos_v, buf0, buf1,
             in_s0, in_s1, out_s0, out_s1):
    bufs = (buf0, buf1)
    in_sems = (in_s0, in_s1)
    out_sems = (out_s0, out_s1)

    wid = lax.axis_index("s") * NC + lax.axis_index("c")
    c0 = wid * CPW
    r0 = wid * RPW

    pltpu.sync_copy(h_hbm, h_v)
    pltpu.sync_copy(w_hbm, w_v)

    lanes = lax.iota(jnp.int32, L)

    def posq(q, carry):
        jj = q // HEIGHT
        h = q % HEIGHT
        cidx = jnp.full((L,), c0 + jj, jnp.int32)
        hvv = plsc.load_gather(h_v, [jnp.full((L,), h, jnp.int32), cidx])
        wv0 = plsc.load_gather(w_v, [lanes, cidx])
        wv1 = plsc.load_gather(w_v, [lanes + L, cidx])
        pos_v[jj, pl.ds(h * WIDTH, L)] = hvv + wv0
        pos_v[jj, pl.ds(h * WIDTH + L, L)] = hvv + wv1
        return carry

    lax.fori_loop(0, CPW * HEIGHT, posq, 0)

    def in_slice(idx):
        return x_hbm.at[pl.ds(idx * CB, CB), pl.ds(r0, RPW), :]

    def out_slice(idx):
        return out_hbm.at[pl.ds(idx * CB, CB), pl.ds(r0, RPW), :]

    for t in range(NBUF):
        pltpu.async_copy(in_slice(t), bufs[t], in_sems[t])

    def round_body(g, carry):
        for t in range(NBUF):
            idx = g * NBUF + t
            buf = bufs[t]
            pltpu.make_async_copy(in_slice(idx), buf, in_sems[t]).wait()

            def ck(h, kcarry, buf=buf):
                for j in range(CPW):
                    for half in range(2):
                        pv = pos_v[j, pl.ds(h * WIDTH + half * L, L)]
                        for b in range(CB):
                            plsc.addupdate(
                                buf.at[b, j * HEIGHT + h,
                                       pl.ds(half * L, L)], pv)
                return kcarry

            lax.fori_loop(0, HEIGHT, ck, 0)

            pltpu.async_copy(buf, out_slice(idx), out_sems[t])

            nxt = idx + PF
            t2 = (t + PF) % NBUF

            @pl.when(jnp.logical_and(nxt >= NBUF, nxt < NCHUNK))
            def _refill(nxt=nxt, t2=t2):
                pltpu.make_async_copy(bufs[t2], out_slice(nxt - NBUF),
                                      out_sems[t2]).wait()
                pltpu.async_copy(in_slice(nxt), bufs[t2], in_sems[t2])

        return carry

    lax.fori_loop(0, NCHUNK // NBUF, round_body, 0)

    for idx in range(NCHUNK - NBUF, NCHUNK):
        t = idx % NBUF
        pltpu.make_async_copy(bufs[t], out_slice(idx), out_sems[t]).wait()


def kernel(x, h_emb, w_emb):
    b, c, h, w = x.shape
    xf = x.reshape(b, c * h, w)

    mesh = plsc.VectorSubcoreMesh(core_axis_name="c", subcore_axis_name="s")
    run = pl.kernel(
        _sc_body,
        mesh=mesh,
        compiler_params=pltpu.CompilerParams(needs_layout_passes=False),
        out_type=jax.ShapeDtypeStruct((b, c * h, w), jnp.float32),
        scratch_types=[
            pltpu.VMEM((HEIGHT, CH), jnp.float32),
            pltpu.VMEM((WIDTH, CH), jnp.float32),
            pltpu.VMEM((CPW, HW), jnp.float32),
            pltpu.VMEM((CB, RPW, WIDTH), jnp.float32),
            pltpu.VMEM((CB, RPW, WIDTH), jnp.float32),
            pltpu.SemaphoreType.DMA,
            pltpu.SemaphoreType.DMA,
            pltpu.SemaphoreType.DMA,
            pltpu.SemaphoreType.DMA,
        ],
    )
    out = run(xf, h_emb, w_emb)
    return out.reshape(b, c, h, w)


# speedup vs baseline: 2.6751x; 2.0751x over previous
import jax
import jax.numpy as jnp
from jax import lax
from jax.experimental import pallas as pl
from jax.experimental.pallas import tpu as pltpu
from jax.experimental.pallas import tpu_sc as plsc

HEIGHT = 32
WIDTH = 32
CH = 192
B = 64
HW = HEIGHT * WIDTH
L = 16

NC = 2
NS = 16
CPW = 8
NWORK = CH // CPW

_NB = 16


def _pos_body(h_hbm, w_hbm, pos_hbm, h_v, w_v, pos_v):
    wid = lax.axis_index("s") * NC + lax.axis_index("c")

    @pl.when(wid < NWORK)
    def _():
        c0 = wid * CPW

        pltpu.sync_copy(h_hbm, h_v)
        pltpu.sync_copy(w_hbm, w_v)

        lanes = lax.iota(jnp.int32, L)

        def posq(q, carry):
            jj = q // HEIGHT
            h = q % HEIGHT
            cidx = jnp.full((L,), c0 + jj, jnp.int32)
            hvv = plsc.load_gather(
                h_v, [jnp.full((L,), h, jnp.int32), cidx])
            wv0 = plsc.load_gather(w_v, [lanes, cidx])
            wv1 = plsc.load_gather(w_v, [lanes + L, cidx])
            pos_v[jj, pl.ds(h * WIDTH, L)] = hvv + wv0
            pos_v[jj, pl.ds(h * WIDTH + L, L)] = hvv + wv1
            return carry

        lax.fori_loop(0, CPW * HEIGHT, posq, 0)
        pltpu.sync_copy(pos_v, pos_hbm.at[pl.ds(c0, CPW), :])


def _add_body(x_ref, pos_ref, o_ref):
    o_ref[...] = x_ref[...] + pos_ref[...][None]


def kernel(x, h_emb, w_emb):
    b, c, h, w = x.shape
    xf = x.reshape(b, c, h * w)

    mesh = plsc.VectorSubcoreMesh(core_axis_name="c", subcore_axis_name="s")
    pos = pl.kernel(
        _pos_body,
        mesh=mesh,
        compiler_params=pltpu.CompilerParams(needs_layout_passes=False),
        out_type=jax.ShapeDtypeStruct((CH, HW), jnp.float32),
        scratch_types=[
            pltpu.VMEM((HEIGHT, CH), jnp.float32),
            pltpu.VMEM((WIDTH, CH), jnp.float32),
            pltpu.VMEM((CPW, HW), jnp.float32),
        ],
    )(h_emb, w_emb)

    out = pl.pallas_call(
        _add_body,
        grid=(b // _NB,),
        in_specs=[
            pl.BlockSpec((_NB, c, h * w), lambda i: (i, 0, 0)),
            pl.BlockSpec((CH, HW), lambda i: (0, 0)),
        ],
        out_specs=pl.BlockSpec((_NB, c, h * w), lambda i: (i, 0, 0)),
        out_shape=jax.ShapeDtypeStruct((b, c, h * w), jnp.float32),
    )(xf, pos)
    return out.reshape(b, c, h, w)
